# baseline (speedup 1.0000x reference)
import functools

import jax
import jax.numpy as jnp
from jax import lax
from jax.experimental import pallas as pl
from jax.experimental.pallas import tpu as pltpu
from jax.experimental.pallas import tpu_sc as plsc

_N = 10000
_D = 128
_E = 320000
_L = 3

_NC = 2
_NS = 16
_NW = _NC * _NS
_CH = 64
_EPT = 10240
_NCH = _EPT // _CH
_GRP = 4
_NGRP = _NCH // _GRP
_PAD = _NW * _EPT - _E
_NPAD = _NS * 640
_ROWS_PT = _NPAD // _NS

_mesh = plsc.VectorSubcoreMesh(core_axis_name="c", subcore_axis_name="s")


@functools.partial(
    pl.kernel,
    out_type=jax.ShapeDtypeStruct((_NC, _NPAD, _D), jnp.float32),
    mesh=_mesh,
    scratch_types=[
        pltpu.VMEM((2 * _GRP, _CH), jnp.int32),
        pltpu.VMEM((2 * _GRP, _CH), jnp.int32),
        pltpu.VMEM((_CH, _D), jnp.float32),
        pltpu.VMEM((_CH, _D), jnp.float32),
        pltpu.VMEM((_CH, _D), jnp.float32),
        pltpu.VMEM((_CH, _D), jnp.float32),
        pltpu.VMEM_SHARED((_NPAD, _D), jnp.float32),
        pltpu.SemaphoreType.DMA,
        pltpu.SemaphoreType.DMA,
        pltpu.SemaphoreType.DMA,
        pltpu.SemaphoreType.DMA,
        pltpu.SemaphoreType.DMA,
        pltpu.SemaphoreType.DMA,
        pltpu.SemaphoreType.DMA,
        pltpu.SemaphoreType.DMA,
        pltpu.SemaphoreType.DMA,
        pltpu.SemaphoreType.DMA,
    ],
)
def _segsum_sc(h_hbm, idx_hbm, out_hbm, ib0, ib1, rb0, rb1, rb2, rb3,
               acc, sg0, sg1, sg2, sg3, ss0, ss1, ss2, ss3, si0, si1):
    rbs = [rb0, rb1, rb2, rb3]
    sgs = [sg0, sg1, sg2, sg3]
    sss = [ss0, ss1, ss2, ss3]
    c = lax.axis_index("c")
    s = lax.axis_index("s")
    w = c * _NS + s

    zvec = jnp.zeros((16,), jnp.float32)

    def _zstore(i, carry):
        rb0[i // (_D // 16), pl.ds((i % (_D // 16)) * 16, 16)] = zvec
        return carry

    lax.fori_loop(0, _CH * (_D // 16), _zstore, 0)

    def _zcopy(i, carry):
        pltpu.sync_copy(rb0, acc.at[pl.ds(s * 640 + i * _CH, _CH)])
        return carry

    lax.fori_loop(0, 640 // _CH, _zcopy, 0)
    plsc.subcore_barrier()

    pltpu.sync_copy(idx_hbm.at[w, 0], ib0)
    for b in range(_GRP):
        pltpu.async_copy(h_hbm.at[ib0.at[2 * b]], rbs[b], sgs[b])
    pltpu.async_copy(idx_hbm.at[w, 1], ib1, si1)

    def _body(gp, carry):
        for parity in range(2):
            g = 2 * gp + parity
            iba, ibb = (ib0, ib1) if parity == 0 else (ib1, ib0)
            sia, sib = (si0, si1) if parity == 0 else (si1, si0)
            for b in range(_GRP):
                pltpu.make_async_copy(
                    h_hbm.at[iba.at[2 * b]], rbs[b], sgs[b]).wait()
                pltpu.async_copy(rbs[b], acc.at[iba.at[2 * b + 1]], sss[b],
                                 add=True)

            @pl.when(g + 1 < _NGRP)
            def _():
                pltpu.make_async_copy(idx_hbm.at[w, g + 1], ibb, sib).wait()
                for b in range(_GRP):
                    pltpu.make_async_copy(
                        rbs[b], acc.at[iba.at[2 * b + 1]], sss[b]).wait()
                    pltpu.async_copy(h_hbm.at[ibb.at[2 * b]], rbs[b], sgs[b])

                @pl.when(g + 2 < _NGRP)
                def _():
                    pltpu.async_copy(idx_hbm.at[w, g + 2], iba, sia)

            @pl.when(g + 1 >= _NGRP)
            def _():
                for b in range(_GRP):
                    pltpu.make_async_copy(
                        rbs[b], acc.at[iba.at[2 * b + 1]], sss[b]).wait()

        return carry

    lax.fori_loop(0, _NGRP // 2, _body, 0)
    plsc.subcore_barrier()
    pltpu.sync_copy(acc.at[pl.ds(s * _ROWS_PT, _ROWS_PT)],
                    out_hbm.at[c, pl.ds(s * _ROWS_PT, _ROWS_PT)])


def _bn(z, gamma, beta, relu):
    mean = jnp.mean(z, axis=0, keepdims=True)
    zc = z - mean
    var = jnp.mean(zc * zc, axis=0, keepdims=True)
    out = gamma * zc * lax.rsqrt(var + 1e-5) + beta
    return jnp.maximum(out, 0.0) if relu else out


def _tc_layer_body(h_ref, p_ref, w1t_ref, w2t_ref, g1_ref, b1_ref, ga_ref,
                   ba_ref, go_ref, bo_ref, out_ref, *, relu_out):
    x = h_ref[...] + p_ref[0, :_N] + p_ref[1, :_N]
    z = jnp.dot(x, w1t_ref[...], preferred_element_type=jnp.float32)
    z = _bn(z, g1_ref[...], b1_ref[...], relu=True)
    z = jnp.dot(z, w2t_ref[...], preferred_element_type=jnp.float32)
    z = _bn(z, ga_ref[...], ba_ref[...], relu=True)
    out_ref[...] = _bn(z, go_ref[...], bo_ref[...], relu=relu_out)


def _tc_layer(relu_out):
    return pl.pallas_call(
        functools.partial(_tc_layer_body, relu_out=relu_out),
        out_shape=jax.ShapeDtypeStruct((_N, _D), jnp.float32),
    )


def kernel(h, edge_index, W1, W2, mlp_bn_gamma, mlp_bn_beta, apply_bn_gamma,
           apply_bn_beta, out_bn_gamma, out_bn_beta):
    src_p = jnp.concatenate(
        [edge_index[0], jnp.zeros((_PAD,), jnp.int32)]).reshape(
            _NW, _NGRP, _GRP, _CH)
    dst_p = jnp.concatenate(
        [edge_index[1], jnp.full((_PAD,), _N, jnp.int32)]).reshape(
            _NW, _NGRP, _GRP, _CH)
    idx = jnp.stack([src_p, dst_p], axis=3).reshape(_NW, _NGRP, 2 * _GRP, _CH)
    for i in range(_L):
        parts = _segsum_sc(h, idx)
        h = _tc_layer(i != _L - 1)(
            h, parts,
            W1[i].T, W2[i].T,
            mlp_bn_gamma[i].reshape(1, _D), mlp_bn_beta[i].reshape(1, _D),
            apply_bn_gamma[i].reshape(1, _D), apply_bn_beta[i].reshape(1, _D),
            out_bn_gamma[i].reshape(1, _D), out_bn_beta[i].reshape(1, _D),
        )
    return h


# speedup vs baseline: 1.0512x; 1.0512x over previous
import functools

import jax
import jax.numpy as jnp
from jax import lax
from jax.experimental import pallas as pl
from jax.experimental.pallas import tpu as pltpu
from jax.experimental.pallas import tpu_sc as plsc

_N = 10000
_D = 128
_E = 320000
_L = 3

_NC = 2
_NS = 16
_NW = _NC * _NS
_CH = 64
_EPT = 10240
_NCH = _EPT // _CH
_NB = 4
_PAD = _NW * _EPT - _E
_NPAD = _NS * 640
_ROWS_PT = _NPAD // _NS

_mesh = plsc.VectorSubcoreMesh(core_axis_name="c", subcore_axis_name="s")


@functools.partial(
    pl.kernel,
    out_type=jax.ShapeDtypeStruct((_NC, _NPAD, _D), jnp.float32),
    mesh=_mesh,
    scratch_types=(
        [pltpu.VMEM((2, _CH), jnp.int32)] * (2 * _NB)
        + [pltpu.VMEM((_CH, _D), jnp.float32)] * _NB
        + [pltpu.VMEM_SHARED((_NPAD, _D), jnp.float32)]
        + [pltpu.SemaphoreType.DMA] * (3 * _NB)
    ),
)
def _segsum_sc(h_hbm, idx_hbm, out_hbm, *refs):
    iba = list(refs[0:_NB])
    ibb = list(refs[_NB:2 * _NB])
    rbs = list(refs[2 * _NB:3 * _NB])
    acc = refs[3 * _NB]
    sgs = list(refs[3 * _NB + 1:4 * _NB + 1])
    sia = list(refs[4 * _NB + 1:5 * _NB + 1])
    sib = list(refs[5 * _NB + 1:6 * _NB + 1])
    rb0 = rbs[0]
    c = lax.axis_index("c")
    s = lax.axis_index("s")
    w = c * _NS + s

    zvec = jnp.zeros((16,), jnp.float32)

    def _zstore(i, carry):
        rb0[i // (_D // 16), pl.ds((i % (_D // 16)) * 16, 16)] = zvec
        return carry

    lax.fori_loop(0, _CH * (_D // 16), _zstore, 0)

    def _zcopy(i, carry):
        pltpu.sync_copy(rb0, acc.at[pl.ds(s * 640 + i * _CH, _CH)])
        return carry

    lax.fori_loop(0, 640 // _CH, _zcopy, 0)
    plsc.subcore_barrier()

    for b in range(_NB):
        pltpu.sync_copy(idx_hbm.at[w, b], iba[b])
        pltpu.async_copy(idx_hbm.at[w, _NB + b], ibb[b], sib[b])
        pltpu.async_copy(h_hbm.at[iba[b].at[0]], rbs[b], sgs[b])

    def _body(i, carry):
        c0 = 2 * _NB * i
        for half in range(2):
            ibs, sis = (iba, sia) if half == 0 else (ibb, sib)
            ibo, sio = (ibb, sib) if half == 0 else (iba, sia)
            for b in range(_NB):
                ch = c0 + half * _NB + b
                pltpu.make_async_copy(
                    h_hbm.at[ibs[b].at[0]], rbs[b], sgs[b]).wait()
                pltpu.sync_copy(rbs[b], acc.at[ibs[b].at[1]], add=True)

                @pl.when(ch + 2 * _NB < _NCH)
                def _():
                    pltpu.async_copy(idx_hbm.at[w, ch + 2 * _NB], ibs[b],
                                     sis[b])

                @pl.when(ch + _NB < _NCH)
                def _():
                    pltpu.make_async_copy(idx_hbm.at[w, ch + _NB], ibo[b],
                                          sio[b]).wait()
                    pltpu.async_copy(h_hbm.at[ibo[b].at[0]], rbs[b], sgs[b])

        return carry

    lax.fori_loop(0, _NCH // (2 * _NB), _body, 0)
    plsc.subcore_barrier()
    pltpu.sync_copy(acc.at[pl.ds(s * _ROWS_PT, _ROWS_PT)],
                    out_hbm.at[c, pl.ds(s * _ROWS_PT, _ROWS_PT)])


def _bn(z, gamma, beta, relu):
    mean = jnp.mean(z, axis=0, keepdims=True)
    zc = z - mean
    var = jnp.mean(zc * zc, axis=0, keepdims=True)
    out = gamma * zc * lax.rsqrt(var + 1e-5) + beta
    return jnp.maximum(out, 0.0) if relu else out


def _tc_layer_body(h_ref, p_ref, w1t_ref, w2t_ref, g1_ref, b1_ref, ga_ref,
                   ba_ref, go_ref, bo_ref, out_ref, *, relu_out):
    x = h_ref[...] + p_ref[0, :_N] + p_ref[1, :_N]
    z = jnp.dot(x, w1t_ref[...], preferred_element_type=jnp.float32)
    z = _bn(z, g1_ref[...], b1_ref[...], relu=True)
    z = jnp.dot(z, w2t_ref[...], preferred_element_type=jnp.float32)
    z = _bn(z, ga_ref[...], ba_ref[...], relu=True)
    out_ref[...] = _bn(z, go_ref[...], bo_ref[...], relu=relu_out)


def _tc_layer(relu_out):
    return pl.pallas_call(
        functools.partial(_tc_layer_body, relu_out=relu_out),
        out_shape=jax.ShapeDtypeStruct((_N, _D), jnp.float32),
    )


def kernel(h, edge_index, W1, W2, mlp_bn_gamma, mlp_bn_beta, apply_bn_gamma,
           apply_bn_beta, out_bn_gamma, out_bn_beta):
    src_p = jnp.concatenate(
        [edge_index[0], jnp.zeros((_PAD,), jnp.int32)]).reshape(
            _NW, _NCH, _CH)
    dst_p = jnp.concatenate(
        [edge_index[1], jnp.full((_PAD,), _N, jnp.int32)]).reshape(
            _NW, _NCH, _CH)
    idx = jnp.stack([src_p, dst_p], axis=2)
    for i in range(_L):
        parts = _segsum_sc(h, idx)
        h = _tc_layer(i != _L - 1)(
            h, parts,
            W1[i].T, W2[i].T,
            mlp_bn_gamma[i].reshape(1, _D), mlp_bn_beta[i].reshape(1, _D),
            apply_bn_gamma[i].reshape(1, _D), apply_bn_beta[i].reshape(1, _D),
            out_bn_gamma[i].reshape(1, _D), out_bn_beta[i].reshape(1, _D),
        )
    return h


# speedup vs baseline: 1.3077x; 1.2440x over previous
import functools

import jax
import jax.numpy as jnp
from jax import lax
from jax.experimental import pallas as pl
from jax.experimental.pallas import tpu as pltpu
from jax.experimental.pallas import tpu_sc as plsc

_N = 10000
_D = 128
_E = 320000
_L = 3

_NC = 2
_NS = 16
_NW = _NC * _NS
_CH = 64
_EPT = 10240
_NCH = _EPT // _CH
_NB = 4
_PPW = _EPT - _E // _NW
_NPAD = _NS * 640
_ROWS_PT = _NPAD // _NS

_mesh = plsc.VectorSubcoreMesh(core_axis_name="c", subcore_axis_name="s")


@functools.partial(
    pl.kernel,
    out_type=jax.ShapeDtypeStruct((_NC, _NPAD, _D), jnp.float32),
    mesh=_mesh,
    scratch_types=(
        [pltpu.VMEM((2, _CH), jnp.int32)] * (2 * _NB)
        + [pltpu.VMEM((_CH, _D), jnp.float32)] * _NB
        + [pltpu.VMEM_SHARED((_NPAD, _D), jnp.float32)]
        + [pltpu.SemaphoreType.DMA] * (3 * _NB)
    ),
)
def _segsum_sc(h_hbm, idx_hbm, out_hbm, *refs):
    iba = list(refs[0:_NB])
    ibb = list(refs[_NB:2 * _NB])
    rbs = list(refs[2 * _NB:3 * _NB])
    acc = refs[3 * _NB]
    sgs = list(refs[3 * _NB + 1:4 * _NB + 1])
    sia = list(refs[4 * _NB + 1:5 * _NB + 1])
    sib = list(refs[5 * _NB + 1:6 * _NB + 1])
    rb0 = rbs[0]
    c = lax.axis_index("c")
    s = lax.axis_index("s")
    w = c * _NS + s

    zvec = jnp.zeros((16,), jnp.float32)

    def _zstore(i, carry):
        rb0[i // (_D // 16), pl.ds((i % (_D // 16)) * 16, 16)] = zvec
        return carry

    lax.fori_loop(0, _CH * (_D // 16), _zstore, 0)

    def _zcopy(i, carry):
        pltpu.sync_copy(rb0, acc.at[pl.ds(s * 640 + i * _CH, _CH)])
        return carry

    lax.fori_loop(0, 640 // _CH, _zcopy, 0)
    plsc.subcore_barrier()

    for b in range(_NB):
        pltpu.sync_copy(idx_hbm.at[w, b], iba[b])
        pltpu.async_copy(idx_hbm.at[w, _NB + b], ibb[b], sib[b])
        pltpu.async_copy(h_hbm.at[iba[b].at[0]], rbs[b], sgs[b])

    def _body(i, carry):
        c0 = 2 * _NB * i
        for half in range(2):
            ibs, sis = (iba, sia) if half == 0 else (ibb, sib)
            ibo, sio = (ibb, sib) if half == 0 else (iba, sia)
            for b in range(_NB):
                ch = c0 + half * _NB + b
                pltpu.make_async_copy(
                    h_hbm.at[ibs[b].at[0]], rbs[b], sgs[b]).wait()
                pltpu.sync_copy(rbs[b], acc.at[ibs[b].at[1]], add=True)

                @pl.when(ch + 2 * _NB < _NCH)
                def _():
                    pltpu.async_copy(idx_hbm.at[w, ch + 2 * _NB], ibs[b],
                                     sis[b])

                @pl.when(ch + _NB < _NCH)
                def _():
                    pltpu.make_async_copy(idx_hbm.at[w, ch + _NB], ibo[b],
                                          sio[b]).wait()
                    pltpu.async_copy(h_hbm.at[ibo[b].at[0]], rbs[b], sgs[b])

        return carry

    lax.fori_loop(0, _NCH // (2 * _NB), _body, 0)
    plsc.subcore_barrier()
    pltpu.sync_copy(acc.at[pl.ds(s * _ROWS_PT, _ROWS_PT)],
                    out_hbm.at[c, pl.ds(s * _ROWS_PT, _ROWS_PT)])


def _bn(z, gamma, beta, relu):
    mean = jnp.mean(z, axis=0, keepdims=True)
    zc = z - mean
    var = jnp.mean(zc * zc, axis=0, keepdims=True)
    out = gamma * zc * lax.rsqrt(var + 1e-5) + beta
    return jnp.maximum(out, 0.0) if relu else out


def _tc_layer_body(h_ref, p_ref, w1t_ref, w2t_ref, g1_ref, b1_ref, ga_ref,
                   ba_ref, go_ref, bo_ref, out_ref, *, relu_out):
    x = h_ref[...] + p_ref[0, :_N] + p_ref[1, :_N]
    z = jnp.dot(x, w1t_ref[...], preferred_element_type=jnp.float32)
    z = _bn(z, g1_ref[...], b1_ref[...], relu=True)
    z = jnp.dot(z, w2t_ref[...], preferred_element_type=jnp.float32)
    z = _bn(z, ga_ref[...], ba_ref[...], relu=True)
    out_ref[...] = _bn(z, go_ref[...], bo_ref[...], relu=relu_out)


def _tc_layer(relu_out):
    return pl.pallas_call(
        functools.partial(_tc_layer_body, relu_out=relu_out),
        out_shape=jax.ShapeDtypeStruct((_N, _D), jnp.float32),
    )


def kernel(h, edge_index, W1, W2, mlp_bn_gamma, mlp_bn_beta, apply_bn_gamma,
           apply_bn_beta, out_bn_gamma, out_bn_beta):
    src_p = jnp.concatenate(
        [edge_index[0].reshape(_NW, _E // _NW),
         jnp.zeros((_NW, _PPW), jnp.int32)], axis=1).reshape(_NW, _NCH, _CH)
    dst_p = jnp.concatenate(
        [edge_index[1].reshape(_NW, _E // _NW),
         jnp.broadcast_to(_N + jnp.arange(_PPW, dtype=jnp.int32),
                          (_NW, _PPW))], axis=1).reshape(_NW, _NCH, _CH)
    idx = jnp.stack([src_p, dst_p], axis=2)
    for i in range(_L):
        parts = _segsum_sc(h, idx)
        h = _tc_layer(i != _L - 1)(
            h, parts,
            W1[i].T, W2[i].T,
            mlp_bn_gamma[i].reshape(1, _D), mlp_bn_beta[i].reshape(1, _D),
            apply_bn_gamma[i].reshape(1, _D), apply_bn_beta[i].reshape(1, _D),
            out_bn_gamma[i].reshape(1, _D), out_bn_beta[i].reshape(1, _D),
        )
    return h
